# baseline (speedup 1.0000x reference)
import functools

import jax
import jax.numpy as jnp
from jax import lax
from jax.experimental import pallas as pl
from jax.experimental.pallas import tpu as pltpu
from jax.experimental.pallas import tpu_sc as plsc

VOCAB = 100000
EMBED = 64
HIDDEN = 128
NUM_CLASSES = 2
B = 4096
L = 200

NC = 2
NS = 16
NW = NC * NS
BPW = B // NW
C1 = 128
C2 = L - C1
NVEC = EMBED // 16
NBUF = 4


def _pool_body(x_hbm, emb_hbm, out_hbm, idx_v, rows_v, pooled_v, *sems):
    wid = lax.axis_index("s") * NC + lax.axis_index("c")
    base = wid * BPW

    pltpu.sync_copy(x_hbm.at[pl.ds(base, BPW)], idx_v)

    def copies(s, b):
        rbuf = rows_v.at[b]
        sem = sems[b]
        h1 = pltpu.make_async_copy(
            emb_hbm.at[idx_v.at[s, pl.ds(0, C1)]], rbuf.at[pl.ds(0, C1)], sem)
        h2 = pltpu.make_async_copy(
            emb_hbm.at[idx_v.at[s, pl.ds(C1, C2)]], rbuf.at[pl.ds(C1, C2)], sem)
        return (h1, h2)

    def fire(s, b):
        for h in copies(s, b):
            h.start()

    def drain(s, b):
        for h in copies(s, b):
            h.wait()

    def accum(s, b):
        rbuf = rows_v.at[b]

        def body(j, accs):
            return tuple(accs[i] + rbuf[j, pl.ds(16 * i, 16)]
                         for i in range(NVEC))

        accs = lax.fori_loop(
            0, L, body,
            tuple(jnp.zeros((16,), jnp.float32) for _ in range(NVEC)),
            unroll=8)
        inv = jnp.float32(1.0 / L)
        for i in range(NVEC):
            pooled_v[s, pl.ds(16 * i, 16)] = accs[i] * inv

    for b in range(NBUF):
        fire(b, b)

    def outer(g, carry):
        for b in range(NBUF):
            s = NBUF * g + b
            drain(s, b)

            @pl.when(s + NBUF < BPW)
            def _():
                fire(s + NBUF, b)

            accum(s, b)
        return carry

    lax.fori_loop(0, BPW // NBUF, outer, 0)

    pltpu.sync_copy(pooled_v, out_hbm.at[pl.ds(base, BPW)])


_pool = pl.kernel(
    _pool_body,
    out_type=jax.ShapeDtypeStruct((B, EMBED), jnp.float32),
    mesh=plsc.VectorSubcoreMesh(core_axis_name="c", subcore_axis_name="s",
                                num_cores=NC, num_subcores=NS),
    scratch_types=[
        pltpu.VMEM((BPW, L), jnp.int32),
        pltpu.VMEM((NBUF, L, EMBED), jnp.float32),
        pltpu.VMEM((BPW, EMBED), jnp.float32),
    ] + [pltpu.SemaphoreType.DMA] * NBUF,
    compiler_params=pltpu.CompilerParams(use_tc_tiling_on_sc=False),
)


DCH = 128
DFULL = VOCAB // DCH
DREM = VOCAB - DFULL * DCH


def _detile_body(embT_hbm, side_hbm, out_hbm, vin, vout, vside):
    wid = lax.axis_index("s") * NC + lax.axis_index("c")
    lanes = lax.iota(jnp.int32, 16)

    def do_chunk(q, nrows):
        i0 = pl.multiple_of(q * DCH, 128)
        o0 = pl.multiple_of(q * (DCH // 2), 8)
        pltpu.sync_copy(embT_hbm.at[:, pl.ds(i0, nrows)],
                        vin.at[:, pl.ds(0, nrows)])

        def rep(jj, carry):
            for h in range(2):
                col = jnp.broadcast_to(2 * jj + h, (16,)).astype(jnp.int32)
                for cc in range(NVEC):
                    g = plsc.load_gather(vin, [lanes + 16 * cc, col])
                    vout[jj, pl.ds(64 * h + 16 * cc, 16)] = g
            return carry

        lax.fori_loop(0, nrows // 2, rep, 0, unroll=2)
        pltpu.sync_copy(vout.at[pl.ds(0, nrows // 2)],
                        out_hbm.at[pl.ds(o0, nrows // 2)])

    def loop(k, carry):
        do_chunk(wid + NW * k, DCH)
        return carry

    lax.fori_loop(0, DFULL // NW, loop, 0)

    @pl.when(wid < DFULL - NW * (DFULL // NW))
    def _():
        do_chunk(NW * (DFULL // NW) + wid, DCH)

    @pl.when(wid == 13)
    def _():
        pltpu.sync_copy(side_hbm, vside)
        nvr = DREM // 2
        for jj in range(nvr):
            for cc in range(2 * EMBED // 16):
                vout[jj, pl.ds(16 * cc, 16)] = (
                    vside[pl.ds(128 * jj + 16 * cc, 16)])
        o0 = (VOCAB // 2) - nvr
        pltpu.sync_copy(vout.at[pl.ds(0, nvr)],
                        out_hbm.at[pl.ds(o0, nvr)])


_detile = pl.kernel(
    _detile_body,
    out_type=jax.ShapeDtypeStruct((VOCAB // 2, 2 * EMBED), jnp.float32),
    mesh=plsc.VectorSubcoreMesh(core_axis_name="c", subcore_axis_name="s",
                                num_cores=NC, num_subcores=NS),
    scratch_types=[
        pltpu.VMEM((EMBED, DCH), jnp.float32),
        pltpu.VMEM((DCH // 2, 2 * EMBED), jnp.float32),
        pltpu.VMEM((DREM * EMBED,), jnp.float32),
    ],
    compiler_params=pltpu.CompilerParams(use_tc_tiling_on_sc=True,
                                         needs_layout_passes=False),
)


def _mlp_body(p_ref, w1_ref, b1_ref, w2_ref, b2_ref, o_ref):
    p = p_ref[:]
    h = lax.dot_general(p, w1_ref[:], (((1,), (1,)), ((), ())),
                        preferred_element_type=jnp.float32)
    h = jnp.maximum(h + b1_ref[:], 0.0)
    o = lax.dot_general(h, w2_ref[:], (((1,), (1,)), ((), ())),
                        preferred_element_type=jnp.float32)
    o_ref[:] = o + b2_ref[:]


_mlp = pl.pallas_call(
    _mlp_body,
    out_shape=jax.ShapeDtypeStruct((B, NUM_CLASSES), jnp.float32),
)


def kernel(x, emb, W1, b1, W2, b2):
    x = x.astype(jnp.int32)
    side = emb[VOCAB - DREM:, :].reshape(DREM * EMBED)
    emb_lin = _detile(emb.T, side).reshape(VOCAB, EMBED)
    pooled = _pool(x, emb_lin)
    return _mlp(pooled, W1, b1.reshape(1, HIDDEN), W2, b2.reshape(1, NUM_CLASSES))


# speedup vs baseline: 1.1298x; 1.1298x over previous
import functools

import jax
import jax.numpy as jnp
from jax import lax
from jax.experimental import pallas as pl
from jax.experimental.pallas import tpu as pltpu
from jax.experimental.pallas import tpu_sc as plsc

VOCAB = 100000
EMBED = 64
HIDDEN = 128
NUM_CLASSES = 2
B = 4096
L = 200

NC = 2
NS = 16
NW = NC * NS
BPW = B // NW
C1 = 128
C2 = L - C1
NVEC = EMBED // 16
NBUF = 4


def _pool_body(x_hbm, emb_hbm, out_hbm, idx_v, rows_v, pooled_v, *sems):
    wid = lax.axis_index("s") * NC + lax.axis_index("c")
    base = wid * BPW

    pltpu.sync_copy(x_hbm.at[pl.ds(base, BPW)], idx_v)

    def copies(s, b):
        rbuf = rows_v.at[b]
        sem = sems[b]
        h1 = pltpu.make_async_copy(
            emb_hbm.at[idx_v.at[s, pl.ds(0, C1)]], rbuf.at[pl.ds(0, C1)], sem)
        h2 = pltpu.make_async_copy(
            emb_hbm.at[idx_v.at[s, pl.ds(C1, C2)]], rbuf.at[pl.ds(C1, C2)], sem)
        return (h1, h2)

    def fire(s, b):
        for h in copies(s, b):
            h.start()

    def drain(s, b):
        for h in copies(s, b):
            h.wait()

    def accum(s, b):
        rbuf = rows_v.at[b]

        def body(j, accs):
            return tuple(accs[i] + rbuf[j, pl.ds(16 * i, 16)]
                         for i in range(NVEC))

        accs = lax.fori_loop(
            0, L, body,
            tuple(jnp.zeros((16,), jnp.float32) for _ in range(NVEC)),
            unroll=8)
        inv = jnp.float32(1.0 / L)
        for i in range(NVEC):
            pooled_v[s, pl.ds(16 * i, 16)] = accs[i] * inv

    for b in range(NBUF):
        fire(b, b)

    def outer(g, carry):
        for b in range(NBUF):
            s = NBUF * g + b
            drain(s, b)

            @pl.when(s + NBUF < BPW)
            def _():
                fire(s + NBUF, b)

            accum(s, b)
        return carry

    lax.fori_loop(0, BPW // NBUF, outer, 0)

    pltpu.sync_copy(pooled_v, out_hbm.at[pl.ds(base, BPW)])


_pool = pl.kernel(
    _pool_body,
    out_type=jax.ShapeDtypeStruct((B, EMBED), jnp.float32),
    mesh=plsc.VectorSubcoreMesh(core_axis_name="c", subcore_axis_name="s",
                                num_cores=NC, num_subcores=NS),
    scratch_types=[
        pltpu.VMEM((BPW, L), jnp.int32),
        pltpu.VMEM((NBUF, L, EMBED), jnp.float32),
        pltpu.VMEM((BPW, EMBED), jnp.float32),
    ] + [pltpu.SemaphoreType.DMA] * NBUF,
    compiler_params=pltpu.CompilerParams(use_tc_tiling_on_sc=False),
)


DCH = 128
DFULL = VOCAB // DCH
DREM = VOCAB - DFULL * DCH


DPAD = 8
KFULL = DFULL // NW
KEXTRA = DFULL - NW * KFULL


def _detile_body(embT_hbm, side_hbm, out_hbm, vin, vout, vside, *sems):
    wid = lax.axis_index("s") * NC + lax.axis_index("c")
    lanes = lax.iota(jnp.int32, 16)
    isem0, isem1, osem0, osem1 = sems
    isems = (isem0, isem1)
    osems = (osem0, osem1)
    nch = KFULL + jnp.where(wid < KEXTRA, 1, 0)

    def chunk_id(k):
        return jnp.where(k < KFULL, wid + NW * k, NW * KFULL + wid)

    def in_copy(k, b):
        i0 = pl.multiple_of(chunk_id(k) * DCH, 128)
        return pltpu.make_async_copy(
            embT_hbm.at[:, pl.ds(i0, DCH)],
            vin.at[b, :, pl.ds(0, DCH)], isems[b])

    def out_copy(k, b):
        o0 = pl.multiple_of(chunk_id(k) * (DCH // 2), 8)
        return pltpu.make_async_copy(
            vout.at[b], out_hbm.at[pl.ds(o0, DCH // 2)], osems[b])

    def fire(k, b):
        @pl.when(k < nch)
        def _():
            in_copy(k, b).start()

    def proc(k, b):
        @pl.when(k < nch)
        def _():
            in_copy(k, b).wait()

            @pl.when(k >= 2)
            def _():
                out_copy(k - 2, b).wait()

            def rep(jj, carry):
                for h in range(2):
                    col = jnp.broadcast_to(2 * jj + h, (16,)).astype(jnp.int32)
                    for cc in range(NVEC):
                        g = plsc.load_gather(vin.at[b], [lanes + 16 * cc, col])
                        vout[b, jj, pl.ds(64 * h + 16 * cc, 16)] = g
                return carry

            lax.fori_loop(0, DCH // 2, rep, 0, unroll=2)
            out_copy(k, b).start()

    fire(0, 0)
    fire(1, 1)

    def outer(g, carry):
        for b in range(2):
            k = 2 * g + b
            proc(k, b)
            fire(k + 2, b)
        return carry

    lax.fori_loop(0, (KFULL + 2) // 2, outer, 0)

    for b in range(2):
        @pl.when(nch - 2 + b >= 0)
        def _(b=b):
            out_copy(nch - 2 + b, b).wait()

    @pl.when(wid == 13)
    def _():
        pltpu.sync_copy(side_hbm, vside)
        nvr = DREM // 2
        for jj in range(nvr):
            for cc in range(2 * EMBED // 16):
                vout[0, jj, pl.ds(16 * cc, 16)] = (
                    vside[pl.ds(128 * jj + 16 * cc, 16)])
        o0 = (VOCAB // 2) - nvr
        pltpu.sync_copy(vout.at[0, pl.ds(0, nvr)],
                        out_hbm.at[pl.ds(o0, nvr)])


_detile = pl.kernel(
    _detile_body,
    out_type=jax.ShapeDtypeStruct((VOCAB // 2, 2 * EMBED), jnp.float32),
    mesh=plsc.VectorSubcoreMesh(core_axis_name="c", subcore_axis_name="s",
                                num_cores=NC, num_subcores=NS),
    scratch_types=[
        pltpu.VMEM((2, EMBED, DCH + DPAD), jnp.float32),
        pltpu.VMEM((2, DCH // 2, 2 * EMBED), jnp.float32),
        pltpu.VMEM((DREM * EMBED,), jnp.float32),
    ] + [pltpu.SemaphoreType.DMA] * 4,
    compiler_params=pltpu.CompilerParams(use_tc_tiling_on_sc=True,
                                         needs_layout_passes=False),
)


def _mlp_body(p_ref, w1_ref, b1_ref, w2_ref, b2_ref, o_ref):
    p = p_ref[:]
    h = lax.dot_general(p, w1_ref[:], (((1,), (1,)), ((), ())),
                        preferred_element_type=jnp.float32)
    h = jnp.maximum(h + b1_ref[:], 0.0)
    o = lax.dot_general(h, w2_ref[:], (((1,), (1,)), ((), ())),
                        preferred_element_type=jnp.float32)
    o_ref[:] = o + b2_ref[:]


_mlp = pl.pallas_call(
    _mlp_body,
    out_shape=jax.ShapeDtypeStruct((B, NUM_CLASSES), jnp.float32),
)


def kernel(x, emb, W1, b1, W2, b2):
    x = x.astype(jnp.int32)
    side = emb[VOCAB - DREM:, :].reshape(DREM * EMBED)
    emb_lin = _detile(emb.T, side).reshape(VOCAB, EMBED)
    pooled = _pool(x, emb_lin)
    return _mlp(pooled, W1, b1.reshape(1, HIDDEN), W2, b2.reshape(1, NUM_CLASSES))


# speedup vs baseline: 1.7751x; 1.5712x over previous
import functools

import jax
import jax.numpy as jnp
from jax import lax
from jax.experimental import pallas as pl
from jax.experimental.pallas import tpu as pltpu
from jax.experimental.pallas import tpu_sc as plsc

VOCAB = 100000
EMBED = 64
HIDDEN = 128
NUM_CLASSES = 2
B = 4096
L = 200

NC = 2
NS = 16
NW = NC * NS
BPW = B // NW
C1 = 128
C2 = L - C1
NVEC = EMBED // 16
NBUF = 4


def _pool_body(x_hbm, emb_hbm, out_hbm, idx_v, rows_v, pooled_v, *sems):
    wid = lax.axis_index("s") * NC + lax.axis_index("c")
    base = wid * BPW

    pltpu.sync_copy(x_hbm.at[pl.ds(base, BPW)], idx_v)

    def copies(s, b):
        rbuf = rows_v.at[b]
        sem = sems[b]
        h1 = pltpu.make_async_copy(
            emb_hbm.at[idx_v.at[s, pl.ds(0, C1)]], rbuf.at[pl.ds(0, C1)], sem)
        h2 = pltpu.make_async_copy(
            emb_hbm.at[idx_v.at[s, pl.ds(C1, C2)]], rbuf.at[pl.ds(C1, C2)], sem)
        return (h1, h2)

    def fire(s, b):
        for h in copies(s, b):
            h.start()

    def drain(s, b):
        for h in copies(s, b):
            h.wait()

    def accum(s, b):
        rbuf = rows_v.at[b]

        def body(j, accs):
            return tuple(accs[i] + rbuf[j, pl.ds(16 * i, 16)]
                         for i in range(NVEC))

        accs = lax.fori_loop(
            0, L, body,
            tuple(jnp.zeros((16,), jnp.float32) for _ in range(NVEC)),
            unroll=8)
        inv = jnp.float32(1.0 / L)
        for i in range(NVEC):
            pooled_v[s, pl.ds(16 * i, 16)] = accs[i] * inv

    for b in range(NBUF):
        fire(b, b)

    def outer(g, carry):
        for b in range(NBUF):
            s = NBUF * g + b
            drain(s, b)

            @pl.when(s + NBUF < BPW)
            def _():
                fire(s + NBUF, b)

            accum(s, b)
        return carry

    lax.fori_loop(0, BPW // NBUF, outer, 0)

    pltpu.sync_copy(pooled_v, out_hbm.at[pl.ds(base, BPW)])


_pool = pl.kernel(
    _pool_body,
    out_type=jax.ShapeDtypeStruct((B, EMBED), jnp.float32),
    mesh=plsc.VectorSubcoreMesh(core_axis_name="c", subcore_axis_name="s",
                                num_cores=NC, num_subcores=NS),
    scratch_types=[
        pltpu.VMEM((BPW, L), jnp.int32),
        pltpu.VMEM((NBUF, L, EMBED), jnp.float32),
        pltpu.VMEM((BPW, EMBED), jnp.float32),
    ] + [pltpu.SemaphoreType.DMA] * NBUF,
    compiler_params=pltpu.CompilerParams(use_tc_tiling_on_sc=False),
)


def _mlp_body(p_ref, w1_ref, b1_ref, w2_ref, b2_ref, o_ref):
    p = p_ref[:]
    h = lax.dot_general(p, w1_ref[:], (((1,), (1,)), ((), ())),
                        preferred_element_type=jnp.float32)
    h = jnp.maximum(h + b1_ref[:], 0.0)
    o = lax.dot_general(h, w2_ref[:], (((1,), (1,)), ((), ())),
                        preferred_element_type=jnp.float32)
    o_ref[:] = o + b2_ref[:]


_mlp = pl.pallas_call(
    _mlp_body,
    out_shape=jax.ShapeDtypeStruct((B, NUM_CLASSES), jnp.float32),
)


def kernel(x, emb, W1, b1, W2, b2):
    x = x.astype(jnp.int32)
    pooled = _pool(x, emb)
    return _mlp(pooled, W1, b1.reshape(1, HIDDEN), W2, b2.reshape(1, NUM_CLASSES))
